# baseline (speedup 1.0000x reference)
import functools

import jax
import jax.numpy as jnp
from jax import lax
from jax.experimental import pallas as pl
from jax.experimental.pallas import tpu as pltpu
from jax.experimental.pallas import tpu_sc as plsc

_A = -0.75

_NC = 2
_NS = 16
_NW = _NC * _NS
_L = 16
_G = 8


def _cubic_weights(t):
    a = _A
    t2 = t * t
    t3 = t2 * t
    w0 = a * (t3 - 2.0 * t2 + t)
    w1 = (a + 2.0) * t3 - (a + 3.0) * t2 + 1.0
    s = 1.0 - t
    s2 = s * s
    s3 = s2 * s
    w2 = (a + 2.0) * s3 - (a + 3.0) * s2 + 1.0
    u = 2.0 - t
    w3 = a * (u * u * u) - 5.0 * a * (u * u) + 8.0 * a * u - 4.0 * a
    return (w0, w1, w2, w3)


@functools.partial(jax.jit, static_argnums=(3, 4, 5, 6))
def _interp(table, xs, ys, H, W, C, n_out):
    npad = xs.shape[0]
    pw = npad // _NW
    chunks = pw // _L
    cw = C // 2
    wcs = cw // _L
    ngroups = pw // _G

    mesh = plsc.VectorSubcoreMesh(core_axis_name="c", subcore_axis_name="s")

    @functools.partial(
        pl.kernel,
        mesh=mesh,
        compiler_params=pltpu.CompilerParams(needs_layout_passes=False),
        out_type=jax.ShapeDtypeStruct((n_out, C), jnp.float32),
        scratch_types=[
            pltpu.VMEM((pw,), jnp.float32),
            pltpu.VMEM((pw,), jnp.float32),
            pltpu.VMEM((_L, _L), jnp.int32),
            pltpu.VMEM((_L, _L), jnp.float32),
            pltpu.VMEM((pw * _L,), jnp.int32),
            pltpu.VMEM((pw * _L,), jnp.float32),
            pltpu.VMEM((_G * _L, cw), jnp.int32),
            pltpu.VMEM((_G * _L, cw), jnp.int32),
            pltpu.VMEM((_L, C), jnp.float32),
            pltpu.SemaphoreType.DMA,
            pltpu.SemaphoreType.DMA,
        ],
    )
    def k(table_hbm, xs_hbm, ys_hbm, out_hbm,
          xs_v, ys_v, idx_t, w_t, idx_pm, w_pm, rows0, rows1, out0,
          sem0, sem1):
        wid = lax.axis_index("s") * _NC + lax.axis_index("c")
        base = wid * pw
        pltpu.sync_copy(xs_hbm.at[pl.ds(base, pw)], xs_v)
        pltpu.sync_copy(ys_hbm.at[pl.ds(base, pw)], ys_v)

        lane = lax.iota(jnp.int32, _L)
        wm1 = float(W - 1)
        hm1 = float(H - 1)

        def phase_a(c, carry):
            off = pl.multiple_of(c * _L, _L)
            px = xs_v[pl.ds(off, _L)]
            py = ys_v[pl.ds(off, _L)]
            gx = 2.0 * (px / wm1) - 1.0
            gy = 2.0 * (py / hm1) - 1.0
            ix = (gx + 1.0) * 0.5 * wm1
            iy = (gy + 1.0) * 0.5 * hm1
            x0i = ix.astype(jnp.int32)
            y0i = iy.astype(jnp.int32)
            tx = ix - x0i.astype(jnp.float32)
            ty = iy - y0i.astype(jnp.float32)
            wx = _cubic_weights(tx)
            wy = _cubic_weights(ty)

            for k_ in range(16):
                j, i = k_ // 4, k_ % 4
                yj = y0i + (j - 1)
                xi = x0i + (i - 1)
                valid = ((yj >= 0) & (yj <= H - 1)
                         & (xi >= 0) & (xi <= W - 1))
                yc = jnp.clip(yj, 0, H - 1)
                xc = jnp.clip(xi, 0, W - 1)
                idx_t[k_, :] = yc * W + xc
                w_t[k_, :] = jnp.where(valid, wy[j] * wx[i],
                                       jnp.zeros_like(tx))

            for p in range(_L):
                col = jnp.full((_L,), p, jnp.int32)
                o = pl.multiple_of(off * _L + p * _L, _L)
                idx_pm[pl.ds(o, _L)] = plsc.load_gather(idx_t, [lane, col])
                w_pm[pl.ds(o, _L)] = plsc.load_gather(w_t, [lane, col])
            return carry

        lax.fori_loop(0, chunks, phase_a, 0)

        def issue(gi, slot_ref, sem):
            o = pl.multiple_of(gi * _G * _L, _G * _L)
            pltpu.async_copy(
                table_hbm.at[idx_pm.at[pl.ds(o, _G * _L)]], slot_ref, sem)

        def wait(gi, slot_ref, sem):
            o = pl.multiple_of(gi * _G * _L, _G * _L)
            pltpu.make_async_copy(
                table_hbm.at[idx_pm.at[pl.ds(o, _G * _L)]], slot_ref,
                sem).wait()

        issue(0, rows0, sem0)
        issue(1, rows1, sem1)
        gpc = _L // _G

        def phase_b(c, carry):
            phase_b_chunk(c, out0)

            @pl.when(base + (c + 1) * _L <= n_out)
            def _():
                pltpu.sync_copy(out0,
                                out_hbm.at[pl.ds(base + c * _L, _L), :])
            return carry

        def phase_b_chunk(c, out_sel):
            for q in range(gpc):
                gi = c * gpc + q
                slot = (rows0, rows1)[q % 2]
                sem = (sem0, sem1)[q % 2]
                wait(gi, slot, sem)
                for lp in range(_G):
                    pf = pl.multiple_of((gi * _G + lp) * _L, _L)
                    wvec = w_pm[pl.ds(pf, _L)]
                    ws = [wvec[t] for t in range(16)]

                    def ccbody(cc, carry3, lp=lp, ws=ws, q=q,
                               slot=slot, out_sel=out_sel):
                        o = pl.multiple_of(cc * _L, _L)
                        v = slot[lp * _L, pl.ds(o, _L)]
                        va = plsc.bitcast(v << 16, jnp.float32)
                        vb = plsc.bitcast(v, jnp.float32)
                        acca = ws[0] * va
                        accb = ws[0] * vb
                        for t in range(1, 16):
                            v = slot[lp * _L + t, pl.ds(o, _L)]
                            va = plsc.bitcast(v << 16, jnp.float32)
                            vb = plsc.bitcast(v, jnp.float32)
                            acca = acca + ws[t] * va
                            accb = accb + ws[t] * vb
                        row = (q * _G + lp) % _L
                        oo = pl.multiple_of(cc * _L, _L)
                        out_sel[row, pl.ds(oo, _L)] = acca
                        out_sel[row, pl.ds(oo + C // 2, _L)] = accb
                        return carry3

                    lax.fori_loop(0, wcs, ccbody, 0, unroll=2)

                @pl.when(gi + 2 < ngroups)
                def _(gi=gi, slot=slot, sem=sem):
                    issue(gi + 2, slot, sem)

        lax.fori_loop(0, chunks, phase_b, 0)

    return k(table, xs, ys)


def kernel(x, pos, H, W):
    C, Hs, Ws = x.shape
    N = pos.shape[0]
    xb = jax.lax.bitcast_convert_type(x.reshape(C, Hs * Ws), jnp.uint32)
    rnd = lambda u: u + jnp.uint32(0x7FFF) + ((u >> 16) & jnp.uint32(1))
    lo = rnd(xb[:C // 2]) >> 16
    hi = rnd(xb[C // 2:]) & jnp.uint32(0xFFFF0000)
    packed = jax.lax.bitcast_convert_type(hi | lo, jnp.int32).T
    per_w = -(-N // (_NW * _L)) * _L
    npad = per_w * _NW
    xs = jnp.pad(pos[:, 0], (0, npad - N))
    ys = jnp.pad(pos[:, 1], (0, npad - N))
    if N % _L == 0:
        return _interp(packed, xs, ys, Hs, Ws, C, N)
    out = _interp(packed, xs, ys, Hs, Ws, C, npad)
    return out[:N]


# speedup vs baseline: 1.4259x; 1.4259x over previous
import functools

import jax
import jax.numpy as jnp
from jax import lax
from jax.experimental import pallas as pl
from jax.experimental.pallas import tpu as pltpu
from jax.experimental.pallas import tpu_sc as plsc

_A = -0.75

_NC = 2
_NS = 16
_NW = _NC * _NS
_L = 16
_G = 8


def _cubic_weights(t):
    a = _A
    t2 = t * t
    t3 = t2 * t
    w0 = a * (t3 - 2.0 * t2 + t)
    w1 = (a + 2.0) * t3 - (a + 3.0) * t2 + 1.0
    s = 1.0 - t
    s2 = s * s
    s3 = s2 * s
    w2 = (a + 2.0) * s3 - (a + 3.0) * s2 + 1.0
    u = 2.0 - t
    w3 = a * (u * u * u) - 5.0 * a * (u * u) + 8.0 * a * u - 4.0 * a
    return (w0, w1, w2, w3)


@functools.partial(jax.jit, static_argnums=(3, 4, 5, 6))
def _interp(table, xs, ys, H, W, C, n_out):
    npad = xs.shape[0]
    pw = npad // _NW
    chunks = pw // _L
    cw = C // 2
    wcs = cw // _L
    ngroups = pw // _G

    mesh = plsc.VectorSubcoreMesh(core_axis_name="c", subcore_axis_name="s")

    @functools.partial(
        pl.kernel,
        mesh=mesh,
        compiler_params=pltpu.CompilerParams(needs_layout_passes=False),
        out_type=jax.ShapeDtypeStruct((n_out, C), jnp.float32),
        scratch_types=[
            pltpu.VMEM((pw,), jnp.float32),
            pltpu.VMEM((pw,), jnp.float32),
            pltpu.VMEM((_L, _L), jnp.int32),
            pltpu.VMEM((_L, _L), jnp.float32),
            pltpu.VMEM((pw * _L,), jnp.int32),
            pltpu.VMEM((pw * _L,), jnp.float32),
            pltpu.VMEM((_G * _L, cw), jnp.int32),
            pltpu.VMEM((_G * _L, cw), jnp.int32),
            pltpu.VMEM((_L, C), jnp.float32),
            pltpu.VMEM((_L, C), jnp.float32),
            pltpu.SemaphoreType.DMA,
            pltpu.SemaphoreType.DMA,
            pltpu.SemaphoreType.DMA,
            pltpu.SemaphoreType.DMA,
        ],
    )
    def k(table_hbm, xs_hbm, ys_hbm, out_hbm,
          xs_v, ys_v, idx_t, w_t, idx_pm, w_pm, rows0, rows1, out0, out1,
          sem0, sem1, osem0, osem1):
        wid = lax.axis_index("s") * _NC + lax.axis_index("c")
        base = wid * pw
        pltpu.sync_copy(xs_hbm.at[pl.ds(base, pw)], xs_v)
        pltpu.sync_copy(ys_hbm.at[pl.ds(base, pw)], ys_v)

        lane = lax.iota(jnp.int32, _L)
        wm1 = float(W - 1)
        hm1 = float(H - 1)

        def phase_a(c, carry):
            off = pl.multiple_of(c * _L, _L)
            px = xs_v[pl.ds(off, _L)]
            py = ys_v[pl.ds(off, _L)]
            gx = 2.0 * (px / wm1) - 1.0
            gy = 2.0 * (py / hm1) - 1.0
            ix = (gx + 1.0) * 0.5 * wm1
            iy = (gy + 1.0) * 0.5 * hm1
            x0i = ix.astype(jnp.int32)
            y0i = iy.astype(jnp.int32)
            tx = ix - x0i.astype(jnp.float32)
            ty = iy - y0i.astype(jnp.float32)
            wx = _cubic_weights(tx)
            wy = _cubic_weights(ty)

            for k_ in range(16):
                j, i = k_ // 4, k_ % 4
                yj = y0i + (j - 1)
                xi = x0i + (i - 1)
                valid = ((yj >= 0) & (yj <= H - 1)
                         & (xi >= 0) & (xi <= W - 1))
                yc = jnp.clip(yj, 0, H - 1)
                xc = jnp.clip(xi, 0, W - 1)
                idx_t[k_, :] = yc * W + xc
                w_t[k_, :] = jnp.where(valid, wy[j] * wx[i],
                                       jnp.zeros_like(tx))

            for p in range(_L):
                col = jnp.full((_L,), p, jnp.int32)
                o = pl.multiple_of(off * _L + p * _L, _L)
                idx_pm[pl.ds(o, _L)] = plsc.load_gather(idx_t, [lane, col])
                w_pm[pl.ds(o, _L)] = plsc.load_gather(w_t, [lane, col])
            return carry

        lax.fori_loop(0, chunks, phase_a, 0)

        def issue(gi, slot_ref, sem):
            o = pl.multiple_of(gi * _G * _L, _G * _L)
            pltpu.async_copy(
                table_hbm.at[idx_pm.at[pl.ds(o, _G * _L)]], slot_ref, sem)

        def wait(gi, slot_ref, sem):
            o = pl.multiple_of(gi * _G * _L, _G * _L)
            pltpu.make_async_copy(
                table_hbm.at[idx_pm.at[pl.ds(o, _G * _L)]], slot_ref,
                sem).wait()

        def out_issue(c, out_slot, osem):
            pltpu.async_copy(
                out_slot, out_hbm.at[pl.ds(base + c * _L, _L), :], osem)

        def out_wait(c, out_slot, osem):
            pltpu.make_async_copy(
                out_slot, out_hbm.at[pl.ds(base + c * _L, _L), :],
                osem).wait()

        issue(0, rows0, sem0)
        issue(1, rows1, sem1)
        gpc = _L // _G

        def phase_b2(h, carry):
            for par in range(2):
                c = h * 2 + par
                out_sel = (out0, out1)[par]
                osem = (osem0, osem1)[par]

                @pl.when((c >= 2) & (base + (c - 1) * _L <= n_out))
                def _(c=c, out_sel=out_sel, osem=osem):
                    out_wait(c - 2, out_sel, osem)

                phase_b_chunk(c, out_sel)

                @pl.when(base + (c + 1) * _L <= n_out)
                def _(c=c, out_sel=out_sel, osem=osem):
                    out_issue(c, out_sel, osem)
            return carry

        def phase_b_chunk(c, out_sel):
            for q in range(gpc):
                gi = c * gpc + q
                slot = (rows0, rows1)[q % 2]
                sem = (sem0, sem1)[q % 2]
                wait(gi, slot, sem)
                for lp in range(_G):
                    pf = pl.multiple_of((gi * _G + lp) * _L, _L)
                    wvec = w_pm[pl.ds(pf, _L)]
                    ws = [wvec[t] for t in range(16)]

                    def ccbody(cc, carry3, lp=lp, ws=ws, q=q,
                               slot=slot, out_sel=out_sel):
                        o = pl.multiple_of(cc * _L, _L)
                        v = slot[lp * _L, pl.ds(o, _L)]
                        va = plsc.bitcast(v << 16, jnp.float32)
                        vb = plsc.bitcast(v, jnp.float32)
                        acca = ws[0] * va
                        accb = ws[0] * vb
                        for t in range(1, 16):
                            v = slot[lp * _L + t, pl.ds(o, _L)]
                            va = plsc.bitcast(v << 16, jnp.float32)
                            vb = plsc.bitcast(v, jnp.float32)
                            acca = acca + ws[t] * va
                            accb = accb + ws[t] * vb
                        row = (q * _G + lp) % _L
                        oo = pl.multiple_of(cc * _L, _L)
                        out_sel[row, pl.ds(oo, _L)] = acca
                        out_sel[row, pl.ds(oo + C // 2, _L)] = accb
                        return carry3

                    lax.fori_loop(0, wcs, ccbody, 0)

                @pl.when(gi + 2 < ngroups)
                def _(gi=gi, slot=slot, sem=sem):
                    issue(gi + 2, slot, sem)

        lax.fori_loop(0, chunks // 2, phase_b2, 0)

        @pl.when(base + (chunks - 1) * _L <= n_out)
        def _():
            out_wait(chunks - 2, out0, osem0)

        @pl.when(base + chunks * _L <= n_out)
        def _():
            out_wait(chunks - 1, out1, osem1)

    return k(table, xs, ys)


def kernel(x, pos, H, W):
    C, Hs, Ws = x.shape
    N = pos.shape[0]
    xb = jax.lax.bitcast_convert_type(x.reshape(C, Hs * Ws), jnp.uint32)
    rnd = lambda u: u + jnp.uint32(0x7FFF) + ((u >> 16) & jnp.uint32(1))
    lo = rnd(xb[:C // 2]) >> 16
    hi = rnd(xb[C // 2:]) & jnp.uint32(0xFFFF0000)
    packed = jax.lax.bitcast_convert_type(hi | lo, jnp.int32).T
    per_w = -(-N // (_NW * _L)) * _L
    npad = per_w * _NW
    xs = jnp.pad(pos[:, 0], (0, npad - N))
    ys = jnp.pad(pos[:, 1], (0, npad - N))
    if N % _L == 0:
        return _interp(packed, xs, ys, Hs, Ws, C, N)
    out = _interp(packed, xs, ys, Hs, Ws, C, npad)
    return out[:N]
